# baseline (speedup 1.0000x reference)
import functools

import jax
import jax.numpy as jnp
import numpy as np
from jax import lax
from jax.experimental import pallas as pl
from jax.experimental.pallas import tpu as pltpu
from jax.experimental.pallas import tpu_sc as plsc

N = 10000
NP = 10112
E = 320000
E2 = E + N
NCH2 = 82
NE = 32 * NCH2 * 128
EPAD = NE - E2
RPS = NP // 16
BN = 1000

_ROW_TAIL = np.concatenate(
    [np.arange(N), (np.arange(EPAD) * 61) % N]).astype(np.int32)
_COL_TAIL = np.concatenate(
    [np.arange(N), N + np.arange(EPAD) % (NP - N)]).astype(np.int32)
_PK_TAIL = (_ROW_TAIL | (_COL_TAIL << 16)).astype(np.int32)
_DEG_TAIL = (N + np.arange(NE - E) % (NP - N)).astype(np.int32)


def _mesh():
    return plsc.VectorSubcoreMesh(core_axis_name="c", subcore_axis_name="s")


def _fill_zeros(zbuf, width):
    for i in range(8):
        for j in range(width // 16):
            zbuf[i, pl.ds(j * 16, 16)] = jnp.zeros((16,), jnp.float32)


def _zero_rows(zbuf, acc, r0):
    def zstep(r, carry):
        pltpu.sync_copy(zbuf, acc.at[pl.ds(r0 + r * 8, 8)])
        return carry

    lax.fori_loop(0, RPS // 8, zstep, 0)


def _edge_loop(xs, pk_v, rc, buf, acc, sem):
    ba = buf.at[pl.ds(0, 128)]
    bb = buf.at[pl.ds(128, 128)]

    def unp(j, r):
        for k in range(8):
            p = pk_v[j, pl.ds(k * 16, 16)]
            rc[r, pl.ds(k * 16, 16)] = lax.bitwise_and(p, 0xFFFF)
            rc[r + 1, pl.ds(k * 16, 16)] = lax.shift_right_logical(p, 16)

    def pair(g, carry):
        j0 = 2 * g
        j1 = j0 + 1

        @pl.when(g > 0)
        def _():
            pltpu.make_async_copy(ba, acc.at[rc.at[1]], sem).wait()
            pltpu.make_async_copy(bb, acc.at[rc.at[3]], sem).wait()

        unp(j0, 0)
        pltpu.async_copy(xs.at[rc.at[0]], ba, sem)
        unp(j1, 2)
        pltpu.async_copy(xs.at[rc.at[2]], bb, sem)
        pltpu.make_async_copy(xs.at[rc.at[0]], ba, sem).wait()
        pltpu.make_async_copy(xs.at[rc.at[2]], bb, sem).wait()
        pltpu.async_copy(ba, acc.at[rc.at[1]], sem, add=True)
        pltpu.async_copy(bb, acc.at[rc.at[3]], sem, add=True)
        return carry

    lax.fori_loop(0, NCH2 // 2, pair, 0)
    pltpu.make_async_copy(ba, acc.at[rc.at[1]], sem).wait()
    pltpu.make_async_copy(bb, acc.at[rc.at[3]], sem).wait()


def _spmm_es_body(xs, pkt, out, pk_v, rc, buf, zbuf, acc, sem):
    c = lax.axis_index("c")
    s = lax.axis_index("s")
    w = s * 2 + c
    pltpu.sync_copy(pkt.at[w], pk_v)
    r0 = s * RPS
    _fill_zeros(zbuf, 128)
    _zero_rows(zbuf, acc, r0)
    plsc.subcore_barrier()
    _edge_loop(xs, pk_v, rc, buf, acc, sem)
    plsc.subcore_barrier()

    @pl.when(c == 0)
    def _():
        pltpu.sync_copy(acc.at[pl.ds(r0, RPS)], out.at[0, pl.ds(r0, RPS)])

    @pl.when(c == 1)
    def _():
        pltpu.sync_copy(acc.at[pl.ds(r0, RPS)], out.at[1, pl.ds(r0, RPS)])


_SPMM_SCRATCH = [
    pltpu.VMEM((NCH2, 128), jnp.int32),
    pltpu.VMEM((4, 128), jnp.int32),
    pltpu.VMEM((256, 128), jnp.float32),
    pltpu.VMEM((8, 128), jnp.float32),
    pltpu.VMEM_SHARED((NP, 128), jnp.float32),
    pltpu.SemaphoreType.DMA,
]

_spmm_es = pl.kernel(
    _spmm_es_body,
    out_type=jax.ShapeDtypeStruct((2, NP, 128), jnp.float32),
    mesh=_mesh(),
    scratch_types=list(_SPMM_SCRATCH),
)


def _spmm2_body(xsa, xsb, pkt, out, pk_v, rc, buf, zbuf, acc, sem):
    c = lax.axis_index("c")
    s = lax.axis_index("s")
    w = s * 2 + c
    pltpu.sync_copy(pkt.at[w], pk_v)
    r0 = s * RPS
    _fill_zeros(zbuf, 128)

    for ph, xs in ((0, xsa), (1, xsb)):
        _zero_rows(zbuf, acc, r0)
        plsc.subcore_barrier()
        _edge_loop(xs, pk_v, rc, buf, acc, sem)
        plsc.subcore_barrier()

        @pl.when(c == 0)
        def _(ph=ph):
            pltpu.sync_copy(acc.at[pl.ds(r0, RPS)],
                            out.at[ph, 0, pl.ds(r0, RPS)])

        @pl.when(c == 1)
        def _(ph=ph):
            pltpu.sync_copy(acc.at[pl.ds(r0, RPS)],
                            out.at[ph, 1, pl.ds(r0, RPS)])


_spmm2 = pl.kernel(
    _spmm2_body,
    out_type=jax.ShapeDtypeStruct((2, 2, NP, 128), jnp.float32),
    mesh=_mesh(),
    scratch_types=list(_SPMM_SCRATCH),
)


def _deg_body(ones_h, colt, out, col_v, buf, zbuf, acc, sem):
    c = lax.axis_index("c")
    s = lax.axis_index("s")
    w = s * 2 + c
    pltpu.sync_copy(colt.at[w], col_v)
    pltpu.sync_copy(ones_h, buf)
    r0 = s * RPS
    _fill_zeros(zbuf, 128)
    _zero_rows(zbuf, acc, r0)
    plsc.subcore_barrier()

    def step(j, carry):
        @pl.when(j > 1)
        def _():
            pltpu.make_async_copy(buf, acc.at[col_v.at[j]], sem).wait()

        pltpu.async_copy(buf, acc.at[col_v.at[j]], sem, add=True)
        return carry

    lax.fori_loop(0, NCH2, step, 0)
    pltpu.make_async_copy(buf, acc.at[col_v.at[0]], sem).wait()
    pltpu.make_async_copy(buf, acc.at[col_v.at[0]], sem).wait()
    plsc.subcore_barrier()

    @pl.when(c == 0)
    def _():
        pltpu.sync_copy(acc.at[pl.ds(r0, RPS)], out.at[0, pl.ds(r0, RPS)])

    @pl.when(c == 1)
    def _():
        pltpu.sync_copy(acc.at[pl.ds(r0, RPS)], out.at[1, pl.ds(r0, RPS)])


_deg_call = pl.kernel(
    _deg_body,
    out_type=jax.ShapeDtypeStruct((2, NP, 128), jnp.float32),
    mesh=_mesh(),
    scratch_types=[
        pltpu.VMEM((NCH2, 128), jnp.int32),
        pltpu.VMEM((128, 128), jnp.float32),
        pltpu.VMEM((8, 128), jnp.float32),
        pltpu.VMEM_SHARED((NP, 128), jnp.float32),
        pltpu.SemaphoreType.DMA,
    ],
)


def _dinv_of(h_ref):
    deg = h_ref[0, :, 0:1] + h_ref[1, :, 0:1] + 1.0
    return lax.rsqrt(deg)


def _tc0_body(ones_ref):
    ones_ref[...] = jnp.ones((128, 128), jnp.float32)


_tc0 = pl.pallas_call(
    _tc0_body,
    grid=(1,),
    in_specs=[],
    out_specs=[pl.BlockSpec((128, 128), lambda i: (0, 0))],
    out_shape=[jax.ShapeDtypeStruct((128, 128), jnp.float32)],
)


def _tc1a_body(x_ref, w1_ref, xw_ref):
    xw_ref[...] = jnp.dot(x_ref[...], w1_ref[...],
                          preferred_element_type=jnp.float32)


_tc1a = pl.pallas_call(
    _tc1a_body,
    grid=(N // BN,),
    in_specs=[
        pl.BlockSpec((BN, 128), lambda i: (i, 0)),
        pl.BlockSpec((128, 256), lambda i: (0, 0)),
    ],
    out_specs=[pl.BlockSpec((BN, 256), lambda i: (i, 0))],
    out_shape=[jax.ShapeDtypeStruct((N, 256), jnp.float32)],
)


def _tc1b_body(h_ref, xw_ref, xsa_ref, xsb_ref):
    dinv = _dinv_of(h_ref)
    xs = xw_ref[...] * dinv
    xsa_ref[...] = xs[:, :128]
    xsb_ref[...] = xs[:, 128:]


def _psum(acc_ref):
    return acc_ref[0] + acc_ref[1]


_tc1b = pl.pallas_call(
    _tc1b_body,
    grid=(N // BN,),
    in_specs=[
        pl.BlockSpec((2, BN, 128), lambda i: (0, i, 0)),
        pl.BlockSpec((BN, 256), lambda i: (i, 0)),
    ],
    out_specs=[pl.BlockSpec((BN, 128), lambda i: (i, 0))] * 2,
    out_shape=[jax.ShapeDtypeStruct((N, 128), jnp.float32)] * 2,
)


def _tc2_body(h_ref, acc1_ref, b1_ref, w2_ref, xs2_ref):
    dinv = _dinv_of(h_ref)
    out1 = jnp.concatenate(
        [acc1_ref[0, 0] + acc1_ref[0, 1], acc1_ref[1, 0] + acc1_ref[1, 1]],
        axis=1) * dinv + b1_ref[...]
    xr = jnp.maximum(out1, 0.0)
    xs2_ref[...] = jnp.dot(xr, w2_ref[...],
                           preferred_element_type=jnp.float32) * dinv


_tc2 = pl.pallas_call(
    _tc2_body,
    grid=(N // BN,),
    in_specs=[
        pl.BlockSpec((2, BN, 128), lambda i: (0, i, 0)),
        pl.BlockSpec((2, 2, BN, 128), lambda i: (0, 0, i, 0)),
        pl.BlockSpec((1, 256), lambda i: (0, 0)),
        pl.BlockSpec((256, 128), lambda i: (0, 0)),
    ],
    out_specs=[pl.BlockSpec((BN, 128), lambda i: (i, 0))],
    out_shape=[jax.ShapeDtypeStruct((N, 128), jnp.float32)],
)


def _tc3_body(h_ref, acc_ref, b2_ref, ew1_ref, eb1_ref, ew2_ref, eb2_ref,
              eps_ref, x2_ref, mu_ref, lv_ref, rx_ref):
    dinv = _dinv_of(h_ref)
    x2 = _psum(acc_ref) * dinv + b2_ref[...]
    t = jnp.maximum(
        jnp.dot(x2, ew1_ref[...], preferred_element_type=jnp.float32) + eb1_ref[...],
        0.0)
    hh = jnp.dot(t, ew2_ref[...], preferred_element_type=jnp.float32) + eb2_ref[...]
    mu = hh[:, :128]
    lv = hh[:, 128:]
    x2_ref[...] = x2
    mu_ref[...] = mu
    lv_ref[...] = lv
    rx_ref[...] = mu + eps_ref[...] * jnp.exp(lv * 0.5)


_tc3 = pl.pallas_call(
    _tc3_body,
    grid=(N // BN,),
    in_specs=[
        pl.BlockSpec((2, BN, 128), lambda i: (0, i, 0)),
        pl.BlockSpec((2, BN, 128), lambda i: (0, i, 0)),
        pl.BlockSpec((1, 128), lambda i: (0, 0)),
        pl.BlockSpec((128, 32), lambda i: (0, 0)),
        pl.BlockSpec((1, 32), lambda i: (0, 0)),
        pl.BlockSpec((32, 256), lambda i: (0, 0)),
        pl.BlockSpec((1, 256), lambda i: (0, 0)),
        pl.BlockSpec((BN, 128), lambda i: (i, 0)),
    ],
    out_specs=[pl.BlockSpec((BN, 128), lambda i: (i, 0))] * 4,
    out_shape=[jax.ShapeDtypeStruct((N, 128), jnp.float32)] * 4,
)


def kernel(x, W1, b1, e1w1, e1b1, e1w2, e1b2, W2, b2, e2w1, e2b1, e2w2, e2b2,
           edge_index):
    ei0 = edge_index[0].astype(jnp.int32)
    ei1 = edge_index[1].astype(jnp.int32)

    pkt2 = jnp.concatenate([ei0 | (ei1 << 16), _PK_TAIL]).reshape(
        32, NCH2, 128)

    degcol = jnp.concatenate([ei0, _DEG_TAIL]).reshape(32, NCH2, 128)

    eps = jax.random.normal(jax.random.fold_in(jax.random.key(1), 2), (N, 128),
                            jnp.float32)

    (ones_t,) = _tc0()
    (xw,) = _tc1a(x, W1)
    dhist = _deg_call(ones_t, degcol)
    xsa, xsb = _tc1b(dhist, xw)
    acc1 = _spmm2(xsa, xsb, pkt2)
    (xs2,) = _tc2(dhist, acc1, b1.reshape(1, 256), W2)
    acc2 = _spmm_es(xs2, pkt2)
    x2, mu, logvar, rx2 = _tc3(dhist, acc2, b2.reshape(1, 128), e2w1,
                               e2b1.reshape(1, 32), e2w2, e2b2.reshape(1, 256),
                               eps)
    return (x2, mu, logvar, rx2)


# speedup vs baseline: 1.0040x; 1.0040x over previous
import jax
import jax.numpy as jnp
import numpy as np
from jax import lax
from jax.experimental import pallas as pl
from jax.experimental.pallas import tpu as pltpu
from jax.experimental.pallas import tpu_sc as plsc

N = 10000
NP = 10112
E = 320000
E2 = E + N
NCH2 = 82
NE = 32 * NCH2 * 128
EPAD = NE - E2
RPS = NP // 16
BN = 1000

_ROW_TAIL = np.concatenate(
    [np.arange(N), (np.arange(EPAD) * 61) % N]).astype(np.int32)
_COL_TAIL = np.concatenate(
    [np.arange(N), N + np.arange(EPAD) % (NP - N)]).astype(np.int32)
_PK_TAIL = (_ROW_TAIL | (_COL_TAIL << 16)).astype(np.int32)
_DEG_TAIL = (N + np.arange(NE - E) % (NP - N)).astype(np.int32)


def _mesh():
    return plsc.VectorSubcoreMesh(core_axis_name="c", subcore_axis_name="s")


def _fill_zeros(zbuf, width):
    for i in range(8):
        for j in range(width // 16):
            zbuf[i, pl.ds(j * 16, 16)] = jnp.zeros((16,), jnp.float32)


def _zero_rows(zbuf, acc, r0):
    def zstep(r, carry):
        pltpu.sync_copy(zbuf, acc.at[pl.ds(r0 + r * 8, 8)])
        return carry

    lax.fori_loop(0, RPS // 8, zstep, 0)


def _edge_loop(xs, pk_v, rc, buf, acc, sem):
    ba = buf.at[pl.ds(0, 128)]
    bb = buf.at[pl.ds(128, 128)]

    def unp(j, r):
        for k in range(8):
            p = pk_v[j, pl.ds(k * 16, 16)]
            rc[r, pl.ds(k * 16, 16)] = lax.bitwise_and(p, 0xFFFF)
            rc[r + 1, pl.ds(k * 16, 16)] = lax.shift_right_logical(p, 16)

    def pair(g, carry):
        j0 = 2 * g
        j1 = j0 + 1

        @pl.when(g > 0)
        def _():
            pltpu.make_async_copy(ba, acc.at[rc.at[1]], sem).wait()
            pltpu.make_async_copy(bb, acc.at[rc.at[3]], sem).wait()

        unp(j0, 0)
        pltpu.async_copy(xs.at[rc.at[0]], ba, sem)
        unp(j1, 2)
        pltpu.async_copy(xs.at[rc.at[2]], bb, sem)
        pltpu.make_async_copy(xs.at[rc.at[0]], ba, sem).wait()
        pltpu.make_async_copy(xs.at[rc.at[2]], bb, sem).wait()
        pltpu.async_copy(ba, acc.at[rc.at[1]], sem, add=True)
        pltpu.async_copy(bb, acc.at[rc.at[3]], sem, add=True)
        return carry

    lax.fori_loop(0, NCH2 // 2, pair, 0)
    pltpu.make_async_copy(ba, acc.at[rc.at[1]], sem).wait()
    pltpu.make_async_copy(bb, acc.at[rc.at[3]], sem).wait()


def _spmm_es_body(xs, pkt, out, pk_v, rc, buf, zbuf, acc, sem):
    c = lax.axis_index("c")
    s = lax.axis_index("s")
    w = s * 2 + c
    pltpu.sync_copy(pkt.at[w], pk_v)
    r0 = s * RPS
    _fill_zeros(zbuf, 128)
    _zero_rows(zbuf, acc, r0)
    plsc.subcore_barrier()
    _edge_loop(xs, pk_v, rc, buf, acc, sem)
    plsc.subcore_barrier()

    @pl.when(c == 0)
    def _():
        pltpu.sync_copy(acc.at[pl.ds(r0, RPS)], out.at[0, pl.ds(r0, RPS)])

    @pl.when(c == 1)
    def _():
        pltpu.sync_copy(acc.at[pl.ds(r0, RPS)], out.at[1, pl.ds(r0, RPS)])


_SPMM_SCRATCH = [
    pltpu.VMEM((NCH2, 128), jnp.int32),
    pltpu.VMEM((4, 128), jnp.int32),
    pltpu.VMEM((256, 128), jnp.float32),
    pltpu.VMEM((8, 128), jnp.float32),
    pltpu.VMEM_SHARED((NP, 128), jnp.float32),
    pltpu.SemaphoreType.DMA,
]

_spmm_es = pl.kernel(
    _spmm_es_body,
    out_type=jax.ShapeDtypeStruct((2, NP, 128), jnp.float32),
    mesh=_mesh(),
    scratch_types=list(_SPMM_SCRATCH),
)


def _spmm2_body(xsa, xsb, pkt, out, pk_v, rc, buf, zbuf, acc, sem):
    c = lax.axis_index("c")
    s = lax.axis_index("s")
    w = s * 2 + c
    pltpu.sync_copy(pkt.at[w], pk_v)
    r0 = s * RPS
    _fill_zeros(zbuf, 128)

    for ph, xs in ((0, xsa), (1, xsb)):
        _zero_rows(zbuf, acc, r0)
        plsc.subcore_barrier()
        _edge_loop(xs, pk_v, rc, buf, acc, sem)
        plsc.subcore_barrier()

        @pl.when(c == 0)
        def _(ph=ph):
            pltpu.sync_copy(acc.at[pl.ds(r0, RPS)],
                            out.at[ph, 0, pl.ds(r0, RPS)])

        @pl.when(c == 1)
        def _(ph=ph):
            pltpu.sync_copy(acc.at[pl.ds(r0, RPS)],
                            out.at[ph, 1, pl.ds(r0, RPS)])


_spmm2 = pl.kernel(
    _spmm2_body,
    out_type=jax.ShapeDtypeStruct((2, 2, NP, 128), jnp.float32),
    mesh=_mesh(),
    scratch_types=list(_SPMM_SCRATCH),
)


def _deg_body(ones_h, colt, out, col_v, buf, zbuf, acc, sem):
    c = lax.axis_index("c")
    s = lax.axis_index("s")
    w = s * 2 + c
    pltpu.sync_copy(colt.at[w], col_v)
    pltpu.sync_copy(ones_h, buf)
    r0 = s * RPS
    _fill_zeros(zbuf, 128)
    _zero_rows(zbuf, acc, r0)
    plsc.subcore_barrier()

    def step(j, carry):
        @pl.when(j > 1)
        def _():
            pltpu.make_async_copy(buf, acc.at[col_v.at[j]], sem).wait()

        pltpu.async_copy(buf, acc.at[col_v.at[j]], sem, add=True)
        return carry

    lax.fori_loop(0, NCH2, step, 0)
    pltpu.make_async_copy(buf, acc.at[col_v.at[0]], sem).wait()
    pltpu.make_async_copy(buf, acc.at[col_v.at[0]], sem).wait()
    plsc.subcore_barrier()

    @pl.when(c == 0)
    def _():
        pltpu.sync_copy(acc.at[pl.ds(r0, RPS)], out.at[0, pl.ds(r0, RPS)])

    @pl.when(c == 1)
    def _():
        pltpu.sync_copy(acc.at[pl.ds(r0, RPS)], out.at[1, pl.ds(r0, RPS)])


_deg_call = pl.kernel(
    _deg_body,
    out_type=jax.ShapeDtypeStruct((2, NP, 128), jnp.float32),
    mesh=_mesh(),
    scratch_types=[
        pltpu.VMEM((NCH2, 128), jnp.int32),
        pltpu.VMEM((128, 128), jnp.float32),
        pltpu.VMEM((8, 128), jnp.float32),
        pltpu.VMEM_SHARED((NP, 128), jnp.float32),
        pltpu.SemaphoreType.DMA,
    ],
)


def _dinv_of(h_ref):
    deg = h_ref[0, :, 0:1] + h_ref[1, :, 0:1] + 1.0
    return lax.rsqrt(deg)


def _tc0_body(ones_ref):
    ones_ref[...] = jnp.ones((128, 128), jnp.float32)


_tc0 = pl.pallas_call(
    _tc0_body,
    grid=(1,),
    in_specs=[],
    out_specs=[pl.BlockSpec((128, 128), lambda i: (0, 0))],
    out_shape=[jax.ShapeDtypeStruct((128, 128), jnp.float32)],
)


def _tc1_body(h_ref, x_ref, w1_ref, xsa_ref, xsb_ref):
    dinv = _dinv_of(h_ref)
    xw = jnp.dot(x_ref[...], w1_ref[...], preferred_element_type=jnp.float32)
    xs = xw * dinv
    xsa_ref[...] = xs[:, :128]
    xsb_ref[...] = xs[:, 128:]


def _psum(acc_ref):
    return acc_ref[0] + acc_ref[1]


_tc1 = pl.pallas_call(
    _tc1_body,
    grid=(N // BN,),
    in_specs=[
        pl.BlockSpec((2, BN, 128), lambda i: (0, i, 0)),
        pl.BlockSpec((BN, 128), lambda i: (i, 0)),
        pl.BlockSpec((128, 256), lambda i: (0, 0)),
    ],
    out_specs=[pl.BlockSpec((BN, 128), lambda i: (i, 0))] * 2,
    out_shape=[jax.ShapeDtypeStruct((N, 128), jnp.float32)] * 2,
)


def _tc2_body(h_ref, acc1_ref, b1_ref, w2_ref, xs2_ref):
    dinv = _dinv_of(h_ref)
    out1 = jnp.concatenate(
        [acc1_ref[0, 0] + acc1_ref[0, 1], acc1_ref[1, 0] + acc1_ref[1, 1]],
        axis=1) * dinv + b1_ref[...]
    xr = jnp.maximum(out1, 0.0)
    xs2_ref[...] = jnp.dot(xr, w2_ref[...],
                           preferred_element_type=jnp.float32) * dinv


_tc2 = pl.pallas_call(
    _tc2_body,
    grid=(N // BN,),
    in_specs=[
        pl.BlockSpec((2, BN, 128), lambda i: (0, i, 0)),
        pl.BlockSpec((2, 2, BN, 128), lambda i: (0, 0, i, 0)),
        pl.BlockSpec((1, 256), lambda i: (0, 0)),
        pl.BlockSpec((256, 128), lambda i: (0, 0)),
    ],
    out_specs=[pl.BlockSpec((BN, 128), lambda i: (i, 0))],
    out_shape=[jax.ShapeDtypeStruct((N, 128), jnp.float32)],
)


def _tc3_body(h_ref, acc_ref, b2_ref, ew1_ref, eb1_ref, ew2_ref, eb2_ref,
              eps_ref, x2_ref, mu_ref, lv_ref, rx_ref):
    dinv = _dinv_of(h_ref)
    x2 = _psum(acc_ref) * dinv + b2_ref[...]
    t = jnp.maximum(
        jnp.dot(x2, ew1_ref[...], preferred_element_type=jnp.float32) + eb1_ref[...],
        0.0)
    hh = jnp.dot(t, ew2_ref[...], preferred_element_type=jnp.float32) + eb2_ref[...]
    mu = hh[:, :128]
    lv = hh[:, 128:]
    x2_ref[...] = x2
    mu_ref[...] = mu
    lv_ref[...] = lv
    rx_ref[...] = mu + eps_ref[...] * jnp.exp(lv * 0.5)


_tc3 = pl.pallas_call(
    _tc3_body,
    grid=(N // BN,),
    in_specs=[
        pl.BlockSpec((2, BN, 128), lambda i: (0, i, 0)),
        pl.BlockSpec((2, BN, 128), lambda i: (0, i, 0)),
        pl.BlockSpec((1, 128), lambda i: (0, 0)),
        pl.BlockSpec((128, 32), lambda i: (0, 0)),
        pl.BlockSpec((1, 32), lambda i: (0, 0)),
        pl.BlockSpec((32, 256), lambda i: (0, 0)),
        pl.BlockSpec((1, 256), lambda i: (0, 0)),
        pl.BlockSpec((BN, 128), lambda i: (i, 0)),
    ],
    out_specs=[pl.BlockSpec((BN, 128), lambda i: (i, 0))] * 4,
    out_shape=[jax.ShapeDtypeStruct((N, 128), jnp.float32)] * 4,
)


def kernel(x, W1, b1, e1w1, e1b1, e1w2, e1b2, W2, b2, e2w1, e2b1, e2w2, e2b2,
           edge_index):
    ei0 = edge_index[0].astype(jnp.int32)
    ei1 = edge_index[1].astype(jnp.int32)

    pkt2 = jnp.concatenate([ei0 | (ei1 << 16), _PK_TAIL]).reshape(
        32, NCH2, 128)

    degcol = jnp.concatenate([ei0, _DEG_TAIL]).reshape(32, NCH2, 128)

    eps = jax.random.normal(jax.random.fold_in(jax.random.key(1), 2), (N, 128),
                            jnp.float32)

    (ones_t,) = _tc0()
    dhist = _deg_call(ones_t, degcol)
    xsa, xsb = _tc1(dhist, x, W1)
    acc1 = _spmm2(xsa, xsb, pkt2)
    (xs2,) = _tc2(dhist, acc1, b1.reshape(1, 256), W2)
    acc2 = _spmm_es(xs2, pkt2)
    x2, mu, logvar, rx2 = _tc3(dhist, acc2, b2.reshape(1, 128), e2w1,
                               e2b1.reshape(1, 32), e2w2, e2b2.reshape(1, 256),
                               eps)
    return (x2, mu, logvar, rx2)


# speedup vs baseline: 1.0176x; 1.0136x over previous
import jax
import jax.numpy as jnp
import numpy as np
from jax import lax
from jax.experimental import pallas as pl
from jax.experimental.pallas import tpu as pltpu
from jax.experimental.pallas import tpu_sc as plsc

N = 10000
NP = 10112
E = 320000
E2 = E + N
NCH2 = 80
NE = 32 * NCH2 * 128
EPAD = NE - E
RPS = NP // 16
BN = 1000

_ROW_TAIL = ((np.arange(EPAD) * 61) % N).astype(np.int32)
_COL_TAIL = (N + np.arange(EPAD) % (NP - N)).astype(np.int32)
_PK_TAIL = (_ROW_TAIL | (_COL_TAIL << 16)).astype(np.int32)
_DEG_TAIL = (N + np.arange(NE - E) % (NP - N)).astype(np.int32)


def _mesh():
    return plsc.VectorSubcoreMesh(core_axis_name="c", subcore_axis_name="s")


def _fill_zeros(zbuf, width):
    for i in range(8):
        for j in range(width // 16):
            zbuf[i, pl.ds(j * 16, 16)] = jnp.zeros((16,), jnp.float32)


def _zero_rows(zbuf, acc, r0):
    def zstep(r, carry):
        pltpu.sync_copy(zbuf, acc.at[pl.ds(r0 + r * 8, 8)])
        return carry

    lax.fori_loop(0, RPS // 8, zstep, 0)


def _edge_loop(xs, pk_v, rc, buf, acc, sem):
    ba = buf.at[pl.ds(0, 128)]
    bb = buf.at[pl.ds(128, 128)]

    def unp(j, r):
        for k in range(8):
            p = pk_v[j, pl.ds(k * 16, 16)]
            rc[r, pl.ds(k * 16, 16)] = lax.bitwise_and(p, 0xFFFF)
            rc[r + 1, pl.ds(k * 16, 16)] = lax.shift_right_logical(p, 16)

    def pair(g, carry):
        j0 = 2 * g
        j1 = j0 + 1

        @pl.when(g > 0)
        def _():
            pltpu.make_async_copy(ba, acc.at[rc.at[1]], sem).wait()
            pltpu.make_async_copy(bb, acc.at[rc.at[3]], sem).wait()

        unp(j0, 0)
        pltpu.async_copy(xs.at[rc.at[0]], ba, sem)
        unp(j1, 2)
        pltpu.async_copy(xs.at[rc.at[2]], bb, sem)
        pltpu.make_async_copy(xs.at[rc.at[0]], ba, sem).wait()
        pltpu.make_async_copy(xs.at[rc.at[2]], bb, sem).wait()
        pltpu.async_copy(ba, acc.at[rc.at[1]], sem, add=True)
        pltpu.async_copy(bb, acc.at[rc.at[3]], sem, add=True)
        return carry

    lax.fori_loop(0, NCH2 // 2, pair, 0)
    pltpu.make_async_copy(ba, acc.at[rc.at[1]], sem).wait()
    pltpu.make_async_copy(bb, acc.at[rc.at[3]], sem).wait()


def _spmm_es_body(xs, pkt, out, pk_v, rc, buf, zbuf, acc, sem):
    c = lax.axis_index("c")
    s = lax.axis_index("s")
    w = s * 2 + c
    pltpu.sync_copy(pkt.at[w], pk_v)
    r0 = s * RPS
    _fill_zeros(zbuf, 128)
    _zero_rows(zbuf, acc, r0)
    plsc.subcore_barrier()
    _edge_loop(xs, pk_v, rc, buf, acc, sem)
    plsc.subcore_barrier()

    @pl.when(c == 0)
    def _():
        pltpu.sync_copy(acc.at[pl.ds(r0, RPS)], out.at[0, pl.ds(r0, RPS)])

    @pl.when(c == 1)
    def _():
        pltpu.sync_copy(acc.at[pl.ds(r0, RPS)], out.at[1, pl.ds(r0, RPS)])


_SPMM_SCRATCH = [
    pltpu.VMEM((NCH2, 128), jnp.int32),
    pltpu.VMEM((4, 128), jnp.int32),
    pltpu.VMEM((256, 128), jnp.float32),
    pltpu.VMEM((8, 128), jnp.float32),
    pltpu.VMEM_SHARED((NP, 128), jnp.float32),
    pltpu.SemaphoreType.DMA,
]

_spmm_es = pl.kernel(
    _spmm_es_body,
    out_type=jax.ShapeDtypeStruct((2, NP, 128), jnp.float32),
    mesh=_mesh(),
    scratch_types=list(_SPMM_SCRATCH),
)


def _spmm2_body(xsa, xsb, pkt, out, pk_v, rc, buf, zbuf, acc, sem):
    c = lax.axis_index("c")
    s = lax.axis_index("s")
    w = s * 2 + c
    pltpu.sync_copy(pkt.at[w], pk_v)
    r0 = s * RPS
    _fill_zeros(zbuf, 128)

    for ph, xs in ((0, xsa), (1, xsb)):
        _zero_rows(zbuf, acc, r0)
        plsc.subcore_barrier()
        _edge_loop(xs, pk_v, rc, buf, acc, sem)
        plsc.subcore_barrier()

        @pl.when(c == 0)
        def _(ph=ph):
            pltpu.sync_copy(acc.at[pl.ds(r0, RPS)],
                            out.at[ph, 0, pl.ds(r0, RPS)])

        @pl.when(c == 1)
        def _(ph=ph):
            pltpu.sync_copy(acc.at[pl.ds(r0, RPS)],
                            out.at[ph, 1, pl.ds(r0, RPS)])


_spmm2 = pl.kernel(
    _spmm2_body,
    out_type=jax.ShapeDtypeStruct((2, 2, NP, 128), jnp.float32),
    mesh=_mesh(),
    scratch_types=list(_SPMM_SCRATCH),
)


def _deg_body(ones_h, colt, out, col_v, buf, zbuf, acc, sem):
    c = lax.axis_index("c")
    s = lax.axis_index("s")
    w = s * 2 + c
    pltpu.sync_copy(colt.at[w], col_v)
    pltpu.sync_copy(ones_h, buf)
    r0 = s * RPS
    _fill_zeros(zbuf, 128)
    _zero_rows(zbuf, acc, r0)
    plsc.subcore_barrier()

    def step(j, carry):
        @pl.when(j > 1)
        def _():
            pltpu.make_async_copy(buf, acc.at[col_v.at[j]], sem).wait()

        pltpu.async_copy(buf, acc.at[col_v.at[j]], sem, add=True)
        return carry

    lax.fori_loop(0, NCH2, step, 0)
    pltpu.make_async_copy(buf, acc.at[col_v.at[0]], sem).wait()
    pltpu.make_async_copy(buf, acc.at[col_v.at[0]], sem).wait()
    plsc.subcore_barrier()

    @pl.when(c == 0)
    def _():
        pltpu.sync_copy(acc.at[pl.ds(r0, RPS)], out.at[0, pl.ds(r0, RPS)])

    @pl.when(c == 1)
    def _():
        pltpu.sync_copy(acc.at[pl.ds(r0, RPS)], out.at[1, pl.ds(r0, RPS)])


_deg_call = pl.kernel(
    _deg_body,
    out_type=jax.ShapeDtypeStruct((2, NP, 128), jnp.float32),
    mesh=_mesh(),
    scratch_types=[
        pltpu.VMEM((NCH2, 128), jnp.int32),
        pltpu.VMEM((128, 128), jnp.float32),
        pltpu.VMEM((8, 128), jnp.float32),
        pltpu.VMEM_SHARED((NP, 128), jnp.float32),
        pltpu.SemaphoreType.DMA,
    ],
)


def _dinv_of(h_ref):
    deg = h_ref[0, :, 0:1] + h_ref[1, :, 0:1] + 1.0
    return lax.rsqrt(deg)


def _tc0_body(ones_ref):
    ones_ref[...] = jnp.ones((128, 128), jnp.float32)


_tc0 = pl.pallas_call(
    _tc0_body,
    grid=(1,),
    in_specs=[],
    out_specs=[pl.BlockSpec((128, 128), lambda i: (0, 0))],
    out_shape=[jax.ShapeDtypeStruct((128, 128), jnp.float32)],
)


def _tc1_body(h_ref, x_ref, w1_ref, xsa_ref, xsb_ref):
    dinv = _dinv_of(h_ref)
    xw = jnp.dot(x_ref[...], w1_ref[...], preferred_element_type=jnp.float32)
    xs = xw * dinv
    xsa_ref[...] = xs[:, :128]
    xsb_ref[...] = xs[:, 128:]


def _psum(acc_ref):
    return acc_ref[0] + acc_ref[1]


_tc1 = pl.pallas_call(
    _tc1_body,
    grid=(N // BN,),
    in_specs=[
        pl.BlockSpec((2, BN, 128), lambda i: (0, i, 0)),
        pl.BlockSpec((BN, 128), lambda i: (i, 0)),
        pl.BlockSpec((128, 256), lambda i: (0, 0)),
    ],
    out_specs=[pl.BlockSpec((BN, 128), lambda i: (i, 0))] * 2,
    out_shape=[jax.ShapeDtypeStruct((N, 128), jnp.float32)] * 2,
)


def _tc2_body(h_ref, acc1_ref, xsa_ref, xsb_ref, b1_ref, w2_ref, xs2_ref):
    dinv = _dinv_of(h_ref)
    accsum = jnp.concatenate(
        [acc1_ref[0, 0] + acc1_ref[0, 1] + xsa_ref[...],
         acc1_ref[1, 0] + acc1_ref[1, 1] + xsb_ref[...]], axis=1)
    out1 = accsum * dinv + b1_ref[...]
    xr = jnp.maximum(out1, 0.0)
    xs2_ref[...] = jnp.dot(xr, w2_ref[...],
                           preferred_element_type=jnp.float32) * dinv


_tc2 = pl.pallas_call(
    _tc2_body,
    grid=(N // BN,),
    in_specs=[
        pl.BlockSpec((2, BN, 128), lambda i: (0, i, 0)),
        pl.BlockSpec((2, 2, BN, 128), lambda i: (0, 0, i, 0)),
        pl.BlockSpec((BN, 128), lambda i: (i, 0)),
        pl.BlockSpec((BN, 128), lambda i: (i, 0)),
        pl.BlockSpec((1, 256), lambda i: (0, 0)),
        pl.BlockSpec((256, 128), lambda i: (0, 0)),
    ],
    out_specs=[pl.BlockSpec((BN, 128), lambda i: (i, 0))],
    out_shape=[jax.ShapeDtypeStruct((N, 128), jnp.float32)],
)


def _tc3_body(h_ref, acc_ref, xs2_ref, b2_ref, ew1_ref, eb1_ref, ew2_ref,
              eb2_ref, eps_ref, x2_ref, mu_ref, lv_ref, rx_ref):
    dinv = _dinv_of(h_ref)
    x2 = (_psum(acc_ref) + xs2_ref[...]) * dinv + b2_ref[...]
    t = jnp.maximum(
        jnp.dot(x2, ew1_ref[...], preferred_element_type=jnp.float32) + eb1_ref[...],
        0.0)
    hh = jnp.dot(t, ew2_ref[...], preferred_element_type=jnp.float32) + eb2_ref[...]
    mu = hh[:, :128]
    lv = hh[:, 128:]
    x2_ref[...] = x2
    mu_ref[...] = mu
    lv_ref[...] = lv
    rx_ref[...] = mu + eps_ref[...] * jnp.exp(lv * 0.5)


_tc3 = pl.pallas_call(
    _tc3_body,
    grid=(N // BN,),
    in_specs=[
        pl.BlockSpec((2, BN, 128), lambda i: (0, i, 0)),
        pl.BlockSpec((2, BN, 128), lambda i: (0, i, 0)),
        pl.BlockSpec((BN, 128), lambda i: (i, 0)),
        pl.BlockSpec((1, 128), lambda i: (0, 0)),
        pl.BlockSpec((128, 32), lambda i: (0, 0)),
        pl.BlockSpec((1, 32), lambda i: (0, 0)),
        pl.BlockSpec((32, 256), lambda i: (0, 0)),
        pl.BlockSpec((1, 256), lambda i: (0, 0)),
        pl.BlockSpec((BN, 128), lambda i: (i, 0)),
    ],
    out_specs=[pl.BlockSpec((BN, 128), lambda i: (i, 0))] * 4,
    out_shape=[jax.ShapeDtypeStruct((N, 128), jnp.float32)] * 4,
)


def kernel(x, W1, b1, e1w1, e1b1, e1w2, e1b2, W2, b2, e2w1, e2b1, e2w2, e2b2,
           edge_index):
    ei0 = edge_index[0].astype(jnp.int32)
    ei1 = edge_index[1].astype(jnp.int32)

    pkt2 = jnp.concatenate([ei0 | (ei1 << 16), _PK_TAIL]).reshape(
        32, NCH2, 128)

    degcol = jnp.concatenate([ei0, _DEG_TAIL]).reshape(32, NCH2, 128)

    eps = jax.random.normal(jax.random.fold_in(jax.random.key(1), 2), (N, 128),
                            jnp.float32)

    (ones_t,) = _tc0()
    dhist = _deg_call(ones_t, degcol)
    xsa, xsb = _tc1(dhist, x, W1)
    acc1 = _spmm2(xsa, xsb, pkt2)
    (xs2,) = _tc2(dhist, acc1, xsa, xsb, b1.reshape(1, 256), W2)
    acc2 = _spmm_es(xs2, pkt2)
    x2, mu, logvar, rx2 = _tc3(dhist, acc2, xs2, b2.reshape(1, 128), e2w1,
                               e2b1.reshape(1, 32), e2w2, e2b2.reshape(1, 256),
                               eps)
    return (x2, mu, logvar, rx2)
